# baseline (speedup 1.0000x reference)
import jax
import jax.numpy as jnp
from jax import lax
from jax.experimental import pallas as pl
from jax.experimental.pallas import tpu as pltpu
from jax.experimental.pallas import tpu_sc as plsc

B, N, D = 32, 576, 384
LANES = 16
NC, NS = 2, 16

QUADS = 8
QROWS = N // QUADS
PER_W = B // 4
SUBS = (40, 32)
NSLAB = PER_W * 2
NB = 6
LA = 3

_sc_mesh = plsc.VectorSubcoreMesh(core_axis_name="c", subcore_axis_name="s")


def _slab(t, m, r0):
    b = m + 4 * (t // 2)
    sub = t % 2
    return b, r0 + (0 if sub == 0 else SUBS[0]), SUBS[sub]


def _sc_fn(x_hbm, p_hbm, o_hbm):
    def scoped(pos_v, bufs, gsems, ssems):
        cid = lax.axis_index("c")
        sid = lax.axis_index("s")
        wid = sid * NC + cid
        q = wid // 4
        m = wid % 4
        r0 = q * QROWS

        pltpu.sync_copy(p_hbm.at[pl.ds(r0, QROWS), :], pos_v)

        def start_gather(t):
            b, rr, rows = _slab(t, m, r0)
            pltpu.make_async_copy(
                x_hbm.at[b, pl.ds(rr, rows), :],
                bufs[t % NB].at[pl.ds(0, rows), :],
                gsems[t % NB],
            ).start()

        def start_scatter(t):
            b, rr, rows = _slab(t, m, r0)
            pltpu.make_async_copy(
                bufs[t % NB].at[pl.ds(0, rows), :],
                o_hbm.at[b, pl.ds(rr, rows), :],
                ssems[t % NB],
            ).start()

        def wait_gather(t):
            _, _, rows = _slab(t, m, r0)
            pltpu.make_async_copy(
                x_hbm.at[0, pl.ds(0, rows), :],
                bufs[t % NB].at[pl.ds(0, rows), :],
                gsems[t % NB],
            ).wait()

        def wait_scatter(t):
            _, _, rows = _slab(t, m, r0)
            pltpu.make_async_copy(
                bufs[t % NB].at[pl.ds(0, rows), :],
                o_hbm.at[0, pl.ds(0, rows), :],
                ssems[t % NB],
            ).wait()

        for t in range(LA):
            start_gather(t)
        for t in range(NSLAB):
            ta = t + LA
            if ta < NSLAB:
                if ta >= NB:
                    wait_scatter(ta - NB)
                start_gather(ta)
            wait_gather(t)

            _, _, rows = _slab(t, m, r0)
            roff = 0 if t % 2 == 0 else SUBS[0]

            @plsc.parallel_loop(0, rows, step=1, unroll=2)
            def _add(r):
                for c in range(D // LANES):
                    s = pl.ds(c * LANES, LANES)
                    plsc.addupdate(
                        bufs[t % NB].at[r, s], pos_v[roff + r, s]
                    )

            start_scatter(t)
        for t in range(NSLAB - NB, NSLAB):
            wait_scatter(t)

    pl.run_scoped(
        scoped,
        pltpu.VMEM((QROWS, D), jnp.float32),
        [pltpu.VMEM((SUBS[0], D), jnp.float32) for _ in range(NB)],
        [pltpu.SemaphoreType.DMA for _ in range(NB)],
        [pltpu.SemaphoreType.DMA for _ in range(NB)],
    )


_sc_add = pl.kernel(
    _sc_fn,
    out_type=jax.ShapeDtypeStruct((B, N, D), jnp.float32),
    mesh=_sc_mesh,
    compiler_params=pltpu.CompilerParams(use_tc_tiling_on_sc=True),
)


def kernel(inputs, pos_table):
    return _sc_add(inputs, pos_table)


# speedup vs baseline: 1.0192x; 1.0192x over previous
import jax
import jax.numpy as jnp
from jax import lax
from jax.experimental import pallas as pl
from jax.experimental.pallas import tpu as pltpu
from jax.experimental.pallas import tpu_sc as plsc

B, N, D = 32, 576, 384
LANES = 16
NC, NS = 2, 16

QUADS = 8
QROWS = N // QUADS
PER_W = B // 4
DC = 128
NDC = D // DC
GROUPS = ((0, 1, 2), (3, 4, 5), (6, 7))
NSLAB = len(GROUPS) * NDC
NBUF = 9

_sc_mesh = plsc.VectorSubcoreMesh(core_axis_name="c", subcore_axis_name="s")


def _slab(s):
    return GROUPS[s // NDC], (s % NDC) * DC, 3 * (s % 3)


def _sc_fn(x_hbm, p_hbm, o_hbm):
    def scoped(pos_v, bufs, gsems, ssems):
        cid = lax.axis_index("c")
        sid = lax.axis_index("s")
        wid = sid * NC + cid
        q = wid // 4
        m = wid % 4
        r0 = q * QROWS

        pltpu.sync_copy(p_hbm.at[pl.ds(r0, QROWS), :], pos_v)

        def start_gathers(s):
            ks, dc, jb = _slab(s)
            for u, k in enumerate(ks):
                pltpu.make_async_copy(
                    x_hbm.at[m + 4 * k, pl.ds(r0, QROWS), pl.ds(dc, DC)],
                    bufs[jb + u],
                    gsems[jb + u],
                ).start()

        def start_scatters(s):
            ks, dc, jb = _slab(s)
            for u, k in enumerate(ks):
                pltpu.make_async_copy(
                    bufs[jb + u],
                    o_hbm.at[m + 4 * k, pl.ds(r0, QROWS), pl.ds(dc, DC)],
                    ssems[jb + u],
                ).start()

        def wait_gathers(s):
            ks, _, jb = _slab(s)
            for u in range(len(ks)):
                pltpu.make_async_copy(
                    x_hbm.at[0, pl.ds(0, QROWS), pl.ds(0, DC)],
                    bufs[jb + u],
                    gsems[jb + u],
                ).wait()

        def wait_scatters(s):
            ks, _, jb = _slab(s)
            for u in range(len(ks)):
                pltpu.make_async_copy(
                    bufs[jb + u],
                    o_hbm.at[0, pl.ds(0, QROWS), pl.ds(0, DC)],
                    ssems[jb + u],
                ).wait()

        start_gathers(0)
        for s in range(NSLAB):
            if s + 1 < NSLAB:
                if s + 1 >= 3:
                    wait_scatters(s - 2)
                start_gathers(s + 1)
            wait_gathers(s)

            ks, dc, jb = _slab(s)
            ng = len(ks)

            @plsc.parallel_loop(0, QROWS, step=1, unroll=2)
            def _add(r):
                for c in range(DC // LANES):
                    v = pos_v[r, pl.ds(dc + c * LANES, LANES)]
                    for u in range(ng):
                        plsc.addupdate(
                            bufs[jb + u].at[r, pl.ds(c * LANES, LANES)], v
                        )

            start_scatters(s)
        for s in range(NSLAB - 3, NSLAB):
            wait_scatters(s)

    pl.run_scoped(
        scoped,
        pltpu.VMEM((QROWS, D), jnp.float32),
        [pltpu.VMEM((QROWS, DC), jnp.float32) for _ in range(NBUF)],
        [pltpu.SemaphoreType.DMA for _ in range(NBUF)],
        [pltpu.SemaphoreType.DMA for _ in range(NBUF)],
    )


_sc_add = pl.kernel(
    _sc_fn,
    out_type=jax.ShapeDtypeStruct((B, N, D), jnp.float32),
    mesh=_sc_mesh,
    compiler_params=pltpu.CompilerParams(use_tc_tiling_on_sc=True),
)


def kernel(inputs, pos_table):
    return _sc_add(inputs, pos_table)
